# baseline (speedup 1.0000x reference)
import functools

import jax
import jax.numpy as jnp
from jax import lax
from jax.experimental import pallas as pl
from jax.experimental.pallas import tpu as pltpu
from jax.experimental.pallas import tpu_sc as plsc

N = 10000
E = 320000
D_FEAT = 128
HIDDEN = 128
NUM_CLASSES = 40

N_PAD = 10240
E_PAD = 327680
ROWS = E_PAD // 128
C_PAD = 64
W = 32

BLK = 2048
GRID = N_PAD // BLK

NC = 2
NS = 16
RPT = N_PAD // NS


def _mesh():
    return plsc.VectorSubcoreMesh(core_axis_name="c", subcore_axis_name="s")


def _rsqrt_deg(d):
    dm = jnp.maximum(d, 1.0)
    i = lax.bitcast_convert_type(dm, jnp.int32)
    i = jnp.int32(0x5F3759DF) - (i >> 1)
    y = lax.bitcast_convert_type(i, jnp.float32)
    for _ in range(3):
        y = y * (1.5 - 0.5 * dm * y * y)
    return jnp.where(d > 0, y, 0.0)


def _make_deg_kernel():
    chunks = ROWS // NS

    @functools.partial(
        pl.kernel,
        mesh=_mesh(),
        compiler_params=pltpu.CompilerParams(use_tc_tiling_on_sc=False),
        out_type=[jax.ShapeDtypeStruct((N_PAD, 16), jnp.float32) for _ in range(2)],
        scratch_types=[
            pltpu.VMEM((chunks, 128), jnp.int32),
            pltpu.VMEM((128, 16), jnp.float32),
            pltpu.VMEM((64, 16), jnp.float32),
            pltpu.VMEM((RPT, 16), jnp.float32),
            pltpu.VMEM_SHARED((N_PAD, 16), jnp.float32),
            pltpu.SemaphoreType.DMA,
        ],
    )
    def deg_kernel(src_hbm, dst_hbm, ns_out, nd_out, ivm, ones_v, zbuf, nbuf,
                   acc, sem):
        c = lax.axis_index("c")
        s = lax.axis_index("s")
        ones16 = jnp.full((16,), 1.0, jnp.float32)
        zeros16 = jnp.zeros((16,), jnp.float32)
        for r in range(128):
            ones_v[r, :] = ones16
        for r in range(64):
            zbuf[r, :] = zeros16

        @pl.when(c == 0)
        def _():
            pltpu.sync_copy(src_hbm.at[pl.ds(s * chunks, chunks)], ivm)

        @pl.when(c == 1)
        def _():
            pltpu.sync_copy(dst_hbm.at[pl.ds(s * chunks, chunks)], ivm)

        for q in range(RPT // 64):
            pltpu.sync_copy(zbuf, acc.at[pl.ds(s * RPT + q * 64, 64)])
        plsc.subcore_barrier()

        def step(j, carry):
            pltpu.async_copy(ones_v, acc.at[ivm.at[j]], sem, add=True)

            @pl.when(j >= 3)
            def _():
                pltpu.make_async_copy(ones_v, acc.at[ivm.at[0]], sem).wait()

            return carry

        lax.fori_loop(0, chunks, step, 0)
        for _ in range(3):
            pltpu.make_async_copy(ones_v, acc.at[ivm.at[0]], sem).wait()
        plsc.subcore_barrier()

        rows = pl.ds(s * RPT, RPT)
        pltpu.sync_copy(acc.at[rows], nbuf)

        def nstep(r, carry):
            nbuf[r, :] = _rsqrt_deg(nbuf[r, :])
            return carry

        lax.fori_loop(0, RPT, nstep, 0)

        @pl.when(c == 0)
        def _():
            pltpu.sync_copy(nbuf, ns_out.at[rows])

        @pl.when(c == 1)
        def _():
            pltpu.sync_copy(nbuf, nd_out.at[rows])

    return deg_kernel


def _make_agg_kernel(phases, nslot, pref):
    NSLOT, PREF = nslot, pref
    chunks = ROWS // NS
    assert chunks % NSLOT == 0 and NSLOT - PREF >= 2

    @functools.partial(
        pl.kernel,
        mesh=_mesh(),
        compiler_params=pltpu.CompilerParams(use_tc_tiling_on_sc=False),
        out_type=[jax.ShapeDtypeStruct((N_PAD, W), jnp.float32)
                  for _ in range(2 * phases)],
        scratch_types=[
            pltpu.VMEM((chunks, 128), jnp.int32),
            pltpu.VMEM((chunks, 128), jnp.int32),
            pltpu.VMEM((NSLOT, 128, W), jnp.float32),
            pltpu.VMEM((32, W), jnp.float32),
            pltpu.VMEM((128, W), jnp.float32),
            pltpu.VMEM((RPT, 16), jnp.float32),
            pltpu.VMEM_SHARED((N_PAD, W), jnp.float32),
            pltpu.VMEM_SHARED((N_PAD, W), jnp.float32),
            pltpu.SemaphoreType.DMA((NSLOT,)),
            pltpu.SemaphoreType.DMA((NSLOT,)),
        ],
    )
    def agg_kernel(*args):
        tables = args[:2 * phases]
        src_hbm, dst_hbm, ns16, nd16 = args[2 * phases:2 * phases + 4]
        outs = args[2 * phases + 4:4 * phases + 4]
        (svm, dvm, gbuf, zbuf, sbuf, nbuf, tspm, acc,
         semg, sems) = args[4 * phases + 4:]
        c = lax.axis_index("c")
        s = lax.axis_index("s")
        zeros16 = jnp.zeros((16,), jnp.float32)
        for r in range(32):
            for q in range(W // 16):
                zbuf[r, pl.ds(q * 16, 16)] = zeros16
        pltpu.sync_copy(src_hbm.at[pl.ds(s * chunks, chunks)], svm)
        pltpu.sync_copy(dst_hbm.at[pl.ds(s * chunks, chunks)], dvm)

        def scale_rows(t):
            def body(r, carry):
                w = nbuf[t * 128 + r, :]
                for q in range(W // 16):
                    sbuf[r, pl.ds(q * 16, 16)] = (
                        sbuf[r, pl.ds(q * 16, 16)] * w)
                return carry

            lax.fori_loop(0, 128, body, 0)

        def run():
            for b in range(PREF):
                pltpu.async_copy(tspm.at[svm.at[b]], gbuf.at[b], semg.at[b])

            def outer(i, carry):
                j0 = i * NSLOT
                for b in range(NSLOT):
                    j = j0 + b
                    pltpu.make_async_copy(
                        tspm.at[svm.at[j]], gbuf.at[b], semg.at[b]).wait()
                    pltpu.async_copy(
                        gbuf.at[b], acc.at[dvm.at[j]], sems.at[b], add=True)
                    bn = (b + PREF) % NSLOT
                    jn = j + PREF

                    @pl.when(jnp.logical_and(jn < chunks, jn >= NSLOT))
                    def _():
                        pltpu.make_async_copy(
                            gbuf.at[bn], acc.at[dvm.at[0]], sems.at[bn]).wait()
                        pltpu.async_copy(
                            tspm.at[svm.at[jn]], gbuf.at[bn], semg.at[bn])

                    @pl.when(jnp.logical_and(jn < chunks, jn < NSLOT))
                    def _():
                        pltpu.async_copy(
                            tspm.at[svm.at[jn]], gbuf.at[bn], semg.at[bn])
                return carry

            lax.fori_loop(0, chunks // NSLOT, outer, 0)
            for b in range(NSLOT):
                pltpu.make_async_copy(
                    gbuf.at[b], acc.at[dvm.at[0]], sems.at[b]).wait()

        for p in range(phases):
            ta, tb = tables[2 * p], tables[2 * p + 1]
            oa, ob = outs[2 * p], outs[2 * p + 1]
            rows = pl.ds(s * RPT, RPT)

            pltpu.sync_copy(ns16.at[rows], nbuf)
            for t in range(RPT // 128):
                r0 = s * RPT + t * 128
                blk = pl.ds(r0, 128)

                @pl.when(c == 0)
                def _():
                    pltpu.sync_copy(ta.at[blk], sbuf)

                @pl.when(c == 1)
                def _():
                    pltpu.sync_copy(tb.at[blk], sbuf)

                scale_rows(t)
                pltpu.sync_copy(sbuf, tspm.at[blk])

            for q in range(RPT // 32):
                pltpu.sync_copy(zbuf, acc.at[pl.ds(s * RPT + q * 32, 32)])
            plsc.subcore_barrier()

            run()
            plsc.subcore_barrier()

            pltpu.sync_copy(nd16.at[rows], nbuf)
            for t in range(RPT // 128):
                r0 = s * RPT + t * 128
                blk = pl.ds(r0, 128)
                pltpu.sync_copy(acc.at[blk], sbuf)
                scale_rows(t)

                @pl.when(c == 0)
                def _():
                    pltpu.sync_copy(sbuf, oa.at[blk])

                @pl.when(c == 1)
                def _():
                    pltpu.sync_copy(sbuf, ob.at[blk])

            if p + 1 < phases:
                plsc.subcore_barrier()

    return agg_kernel


def _tc_b(feat_p, w1):
    def body(feat_ref, w1_ref, h0_ref, h1_ref, h2_ref, h3_ref):
        h = jnp.dot(feat_ref[...], w1_ref[...],
                    preferred_element_type=jnp.float32)
        h0_ref[...] = h[:, 0:32]
        h1_ref[...] = h[:, 32:64]
        h2_ref[...] = h[:, 64:96]
        h3_ref[...] = h[:, 96:128]

    return pl.pallas_call(
        body,
        grid=(GRID,),
        in_specs=[
            pl.BlockSpec((BLK, D_FEAT), lambda i: (i, 0)),
            pl.BlockSpec((D_FEAT, HIDDEN), lambda i: (0, 0)),
        ],
        out_specs=[pl.BlockSpec((BLK, 32), lambda i: (i, 0))] * 4,
        out_shape=[jax.ShapeDtypeStruct((N_PAD, 32), jnp.float32)] * 4,
    )(feat_p, w1)


def _tc_d(a0, a1, a2, a3, w2p, b1r):
    def body(a0_ref, a1_ref, a2_ref, a3_ref, w2_ref, b1_ref, oa_ref, ob_ref):
        x = jnp.concatenate(
            [a0_ref[...], a1_ref[...], a2_ref[...], a3_ref[...]], axis=1)
        x = jax.nn.relu(x + b1_ref[...])
        y = jnp.dot(x, w2_ref[...], preferred_element_type=jnp.float32)
        oa_ref[...] = y[:, :32]
        ob_ref[...] = y[:, 32:]

    return pl.pallas_call(
        body,
        grid=(GRID,),
        in_specs=[pl.BlockSpec((BLK, 32), lambda i: (i, 0))] * 4
        + [
            pl.BlockSpec((HIDDEN, C_PAD), lambda i: (0, 0)),
            pl.BlockSpec((1, HIDDEN), lambda i: (0, 0)),
        ],
        out_specs=[
            pl.BlockSpec((BLK, 32), lambda i: (i, 0)),
            pl.BlockSpec((BLK, 32), lambda i: (i, 0)),
        ],
        out_shape=[
            jax.ShapeDtypeStruct((N_PAD, 32), jnp.float32),
            jax.ShapeDtypeStruct((N_PAD, 32), jnp.float32),
        ],
    )(a0, a1, a2, a3, w2p, b1r)


def _tc_f(ga, gb, b2r):
    def body(ga_ref, gb_ref, b2_ref, out_ref):
        z = jnp.concatenate([ga_ref[...], gb_ref[...]], axis=1) + b2_ref[...]
        col = lax.broadcasted_iota(jnp.int32, (BLK, C_PAD), 1)
        zm = jnp.where(col < NUM_CLASSES, z, -jnp.inf)
        mx = jnp.max(zm, axis=1, keepdims=True)
        e = jnp.where(col < NUM_CLASSES, jnp.exp(zm - mx), 0.0)
        out = zm - mx - jnp.log(jnp.sum(e, axis=1, keepdims=True))
        out_ref[...] = out[:, :NUM_CLASSES]

    return pl.pallas_call(
        body,
        grid=(GRID,),
        in_specs=[
            pl.BlockSpec((BLK, 32), lambda i: (i, 0)),
            pl.BlockSpec((BLK, 32), lambda i: (i, 0)),
            pl.BlockSpec((1, C_PAD), lambda i: (0, 0)),
        ],
        out_specs=pl.BlockSpec((BLK, NUM_CLASSES), lambda i: (i, 0)),
        out_shape=jax.ShapeDtypeStruct((N_PAD, NUM_CLASSES), jnp.float32),
    )(ga, gb, b2r)


_deg_call = _make_deg_kernel()
_agg2ph = _make_agg_kernel(phases=2, nslot=8, pref=6)
_agg1ph = _make_agg_kernel(phases=1, nslot=8, pref=6)


@jax.jit
def kernel(feat, edge_index, W1, b1, W2, b2):
    src = edge_index[0].astype(jnp.int32)
    dst = edge_index[1].astype(jnp.int32)
    pad = jnp.full((E_PAD - E,), N, jnp.int32)
    src3 = jnp.concatenate([src, pad]).reshape(ROWS, 128)
    dst3 = jnp.concatenate([dst, pad]).reshape(ROWS, 128)
    feat_p = jnp.pad(feat, ((0, N_PAD - N), (0, 0)))
    w2p = jnp.pad(W2, ((0, 0), (0, C_PAD - NUM_CLASSES)))
    b1r = b1.reshape(1, HIDDEN)
    b2r = jnp.pad(b2, (0, C_PAD - NUM_CLASSES)).reshape(1, C_PAD)

    ns16, nd16 = _deg_call(src3, dst3)
    h0, h1, h2, h3 = _tc_b(feat_p, W1)
    a0, a1, a2, a3 = _agg2ph(h0, h1, h2, h3, src3, dst3, ns16, nd16)
    h2a, h2b = _tc_d(a0, a1, a2, a3, w2p, b1r)
    ga, gb = _agg1ph(h2a, h2b, src3, dst3, ns16, nd16)
    out = _tc_f(ga, gb, b2r)
    return out[:N]


# speedup vs baseline: 1.1343x; 1.1343x over previous
import functools

import jax
import jax.numpy as jnp
from jax import lax
from jax.experimental import pallas as pl
from jax.experimental.pallas import tpu as pltpu
from jax.experimental.pallas import tpu_sc as plsc

N = 10000
E = 320000
D_FEAT = 128
HIDDEN = 128
NUM_CLASSES = 40

N_PAD = 10240
E_PAD = 327680
ROWS = E_PAD // 128
C_PAD = 64
W = 32

BLK = 2048
GRID = N_PAD // BLK

NC = 2
NS = 16
RPT = N_PAD // NS


def _mesh():
    return plsc.VectorSubcoreMesh(core_axis_name="c", subcore_axis_name="s")


def _make_deg_kernel():
    chunks = ROWS // (NC * NS)

    @functools.partial(
        pl.kernel,
        mesh=_mesh(),
        compiler_params=pltpu.CompilerParams(use_tc_tiling_on_sc=False),
        out_type=[jax.ShapeDtypeStruct((N_PAD, 16), jnp.float32) for _ in range(4)],
        scratch_types=[
            pltpu.VMEM((chunks, 128), jnp.int32),
            pltpu.VMEM((chunks, 128), jnp.int32),
            pltpu.VMEM((128, 16), jnp.float32),
            pltpu.VMEM((64, 16), jnp.float32),
            pltpu.VMEM_SHARED((N_PAD, 16), jnp.float32),
            pltpu.VMEM_SHARED((N_PAD, 16), jnp.float32),
            pltpu.SemaphoreType.DMA,
            pltpu.SemaphoreType.DMA,
        ],
    )
    def deg_kernel(src_hbm, dst_hbm, sa, sb, da, db, svm, dvm, ones_v, zbuf,
                   acc_s, acc_d, sem_a, sem_b):
        c = lax.axis_index("c")
        s = lax.axis_index("s")
        w = s * NC + c
        ones16 = jnp.full((16,), 1.0, jnp.float32)
        zeros16 = jnp.zeros((16,), jnp.float32)
        for r in range(128):
            ones_v[r, :] = ones16
        for r in range(64):
            zbuf[r, :] = zeros16
        pltpu.sync_copy(src_hbm.at[pl.ds(w * chunks, chunks)], svm)
        pltpu.sync_copy(dst_hbm.at[pl.ds(w * chunks, chunks)], dvm)
        for q in range(RPT // 64):
            pltpu.sync_copy(zbuf, acc_s.at[pl.ds(s * RPT + q * 64, 64)])
            pltpu.sync_copy(zbuf, acc_d.at[pl.ds(s * RPT + q * 64, 64)])
        plsc.subcore_barrier()

        def step(j, carry):
            pltpu.async_copy(ones_v, acc_s.at[svm.at[j]], sem_a, add=True)
            pltpu.async_copy(ones_v, acc_d.at[dvm.at[j]], sem_b, add=True)

            @pl.when(j >= 3)
            def _():
                pltpu.make_async_copy(ones_v, acc_s.at[svm.at[0]], sem_a).wait()
                pltpu.make_async_copy(ones_v, acc_d.at[dvm.at[0]], sem_b).wait()

            return carry

        lax.fori_loop(0, chunks, step, 0)
        for _ in range(3):
            pltpu.make_async_copy(ones_v, acc_s.at[svm.at[0]], sem_a).wait()
            pltpu.make_async_copy(ones_v, acc_d.at[dvm.at[0]], sem_b).wait()
        plsc.subcore_barrier()

        @pl.when(c == 0)
        def _():
            pltpu.sync_copy(acc_s.at[pl.ds(s * RPT, RPT)], sa.at[pl.ds(s * RPT, RPT)])
            pltpu.sync_copy(acc_d.at[pl.ds(s * RPT, RPT)], da.at[pl.ds(s * RPT, RPT)])

        @pl.when(c == 1)
        def _():
            pltpu.sync_copy(acc_s.at[pl.ds(s * RPT, RPT)], sb.at[pl.ds(s * RPT, RPT)])
            pltpu.sync_copy(acc_d.at[pl.ds(s * RPT, RPT)], db.at[pl.ds(s * RPT, RPT)])

    return deg_kernel


def _make_agg_kernel(phases, nslot, pref):
    NSLOT, PREF = nslot, pref
    chunks = ROWS // NS
    assert chunks % NSLOT == 0 and NSLOT - PREF >= 2

    @functools.partial(
        pl.kernel,
        mesh=_mesh(),
        compiler_params=pltpu.CompilerParams(use_tc_tiling_on_sc=False),
        out_type=jax.ShapeDtypeStruct((N_PAD, 128), jnp.float32),
        scratch_types=[
            pltpu.VMEM((chunks, 128), jnp.int32),
            pltpu.VMEM((chunks, 128), jnp.int32),
            pltpu.VMEM((NSLOT, 128, W), jnp.float32),
            pltpu.VMEM((64, W), jnp.float32),
            pltpu.VMEM_SHARED((N_PAD, W), jnp.float32),
            pltpu.VMEM_SHARED((N_PAD, W), jnp.float32),
            pltpu.SemaphoreType.DMA((NSLOT,)),
            pltpu.SemaphoreType.DMA((NSLOT,)),
        ],
    )
    def agg_kernel(tbl, src_hbm, dst_hbm, out,
                   svm, dvm, gbuf, zbuf, tspm, acc, semg, sems):
        c = lax.axis_index("c")
        s = lax.axis_index("s")
        zeros16 = jnp.zeros((16,), jnp.float32)
        for r in range(64):
            for q in range(W // 16):
                zbuf[r, pl.ds(q * 16, 16)] = zeros16
        pltpu.sync_copy(src_hbm.at[pl.ds(s * chunks, chunks)], svm)
        pltpu.sync_copy(dst_hbm.at[pl.ds(s * chunks, chunks)], dvm)

        def run():
            for b in range(PREF):
                pltpu.async_copy(tspm.at[svm.at[b]], gbuf.at[b], semg.at[b])

            def outer(i, carry):
                j0 = i * NSLOT
                for b in range(NSLOT):
                    j = j0 + b
                    pltpu.make_async_copy(
                        tspm.at[svm.at[j]], gbuf.at[b], semg.at[b]).wait()
                    pltpu.async_copy(
                        gbuf.at[b], acc.at[dvm.at[j]], sems.at[b], add=True)
                    bn = (b + PREF) % NSLOT
                    jn = j + PREF

                    @pl.when(jnp.logical_and(jn < chunks, jn >= NSLOT))
                    def _():
                        pltpu.make_async_copy(
                            gbuf.at[bn], acc.at[dvm.at[0]], sems.at[bn]).wait()
                        pltpu.async_copy(
                            tspm.at[svm.at[jn]], gbuf.at[bn], semg.at[bn])

                    @pl.when(jnp.logical_and(jn < chunks, jn < NSLOT))
                    def _():
                        pltpu.async_copy(
                            tspm.at[svm.at[jn]], gbuf.at[bn], semg.at[bn])
                return carry

            lax.fori_loop(0, chunks // NSLOT, outer, 0)
            for b in range(NSLOT):
                pltpu.make_async_copy(
                    gbuf.at[b], acc.at[dvm.at[0]], sems.at[b]).wait()

        rows = pl.ds(s * RPT, RPT)
        for p in range(phases):
            cols = pl.ds(p * 64 + c * 32, 32)

            pltpu.sync_copy(tbl.at[rows, cols], tspm.at[rows])
            for q in range(RPT // 64):
                pltpu.sync_copy(zbuf, acc.at[pl.ds(s * RPT + q * 64, 64)])
            plsc.subcore_barrier()

            run()
            plsc.subcore_barrier()

            pltpu.sync_copy(acc.at[rows], out.at[rows, cols])

            if p + 1 < phases:
                plsc.subcore_barrier()

    return agg_kernel


def _tc_b(feat_p, w1, sa, sb, da, db):
    def body(feat_ref, w1_ref, sa_ref, sb_ref, da_ref, db_ref,
             h_ref, ns_ref, nd_ref):
        degs = sa_ref[:, 0:1] + sb_ref[:, 0:1]
        degd = da_ref[:, 0:1] + db_ref[:, 0:1]
        ns = jnp.where(degs > 0, lax.rsqrt(jnp.maximum(degs, 1.0)), 0.0)
        nd = jnp.where(degd > 0, lax.rsqrt(jnp.maximum(degd, 1.0)), 0.0)
        h_ref[...] = jnp.dot(feat_ref[...], w1_ref[...],
                             preferred_element_type=jnp.float32) * ns
        ns_ref[...] = jnp.broadcast_to(ns, (BLK, 8))
        nd_ref[...] = jnp.broadcast_to(nd, (BLK, 8))

    return pl.pallas_call(
        body,
        grid=(GRID,),
        in_specs=[
            pl.BlockSpec((BLK, D_FEAT), lambda i: (i, 0)),
            pl.BlockSpec((D_FEAT, HIDDEN), lambda i: (0, 0)),
            pl.BlockSpec((BLK, 16), lambda i: (i, 0)),
            pl.BlockSpec((BLK, 16), lambda i: (i, 0)),
            pl.BlockSpec((BLK, 16), lambda i: (i, 0)),
            pl.BlockSpec((BLK, 16), lambda i: (i, 0)),
        ],
        out_specs=[pl.BlockSpec((BLK, HIDDEN), lambda i: (i, 0))]
        + [pl.BlockSpec((BLK, 8), lambda i: (i, 0))] * 2,
        out_shape=[jax.ShapeDtypeStruct((N_PAD, HIDDEN), jnp.float32)]
        + [jax.ShapeDtypeStruct((N_PAD, 8), jnp.float32)] * 2,
    )(feat_p, w1, sa, sb, da, db)


def _tc_d(agg1, ns, nd, w2p, b1r):
    def body(a_ref, ns_ref, nd_ref, w2_ref, b1_ref, o_ref):
        x = jax.nn.relu(a_ref[...] * nd_ref[:, 0:1] + b1_ref[...])
        y = jnp.dot(x, w2_ref[...], preferred_element_type=jnp.float32)
        o_ref[...] = y * ns_ref[:, 0:1]

    return pl.pallas_call(
        body,
        grid=(GRID,),
        in_specs=[
            pl.BlockSpec((BLK, HIDDEN), lambda i: (i, 0)),
            pl.BlockSpec((BLK, 8), lambda i: (i, 0)),
            pl.BlockSpec((BLK, 8), lambda i: (i, 0)),
            pl.BlockSpec((HIDDEN, 128), lambda i: (0, 0)),
            pl.BlockSpec((1, HIDDEN), lambda i: (0, 0)),
        ],
        out_specs=pl.BlockSpec((BLK, 128), lambda i: (i, 0)),
        out_shape=jax.ShapeDtypeStruct((N_PAD, 128), jnp.float32),
    )(agg1, ns, nd, w2p, b1r)


def _tc_f(agg2, nd, b2r):
    def body(a_ref, nd_ref, b2_ref, out_ref):
        z = a_ref[:, :C_PAD] * nd_ref[:, 0:1] + b2_ref[...]
        col = lax.broadcasted_iota(jnp.int32, (BLK, C_PAD), 1)
        zm = jnp.where(col < NUM_CLASSES, z, -jnp.inf)
        mx = jnp.max(zm, axis=1, keepdims=True)
        e = jnp.where(col < NUM_CLASSES, jnp.exp(zm - mx), 0.0)
        out = zm - mx - jnp.log(jnp.sum(e, axis=1, keepdims=True))
        out_ref[...] = out[:, :NUM_CLASSES]

    return pl.pallas_call(
        body,
        grid=(GRID,),
        in_specs=[
            pl.BlockSpec((BLK, 128), lambda i: (i, 0)),
            pl.BlockSpec((BLK, 8), lambda i: (i, 0)),
            pl.BlockSpec((1, C_PAD), lambda i: (0, 0)),
        ],
        out_specs=pl.BlockSpec((BLK, NUM_CLASSES), lambda i: (i, 0)),
        out_shape=jax.ShapeDtypeStruct((N_PAD, NUM_CLASSES), jnp.float32),
    )(agg2, nd, b2r)


_deg_call = _make_deg_kernel()
_agg2ph = _make_agg_kernel(phases=2, nslot=8, pref=6)
_agg1ph = _make_agg_kernel(phases=1, nslot=8, pref=6)


@jax.jit
def kernel(feat, edge_index, W1, b1, W2, b2):
    src = edge_index[0].astype(jnp.int32)
    dst = edge_index[1].astype(jnp.int32)
    pad = jnp.full((E_PAD - E,), N, jnp.int32)
    src3 = jnp.concatenate([src, pad]).reshape(ROWS, 128)
    dst3 = jnp.concatenate([dst, pad]).reshape(ROWS, 128)
    feat_p = jnp.pad(feat, ((0, N_PAD - N), (0, 0)))
    w2p = jnp.pad(W2, ((0, 0), (0, 128 - NUM_CLASSES)))
    b1r = b1.reshape(1, HIDDEN)
    b2r = jnp.pad(b2, (0, C_PAD - NUM_CLASSES)).reshape(1, C_PAD)

    sa, sb, da, db = _deg_call(src3, dst3)
    h, ns, nd = _tc_b(feat_p, W1, sa, sb, da, db)
    agg1 = _agg2ph(h, src3, dst3)
    h2 = _tc_d(agg1, ns, nd, w2p, b1r)
    agg2 = _agg1ph(h2, src3, dst3)
    out = _tc_f(agg2, nd, b2r)
    return out[:N]
